# Initial kernel scaffold; baseline (speedup 1.0000x reference)
import jax
import jax.numpy as jnp
from jax.experimental import pallas as pl


def kernel(x, edge_index, edge_attr, u, batch, W1, b1, W2, b2):
    raise NotImplementedError("write your pallas kernel here")


# speedup vs baseline: 7.6513x; 7.6513x over previous
import functools

import jax
import jax.numpy as jnp
from jax import lax
from jax.experimental import pallas as pl
from jax.experimental.pallas import tpu as pltpu
from jax.experimental.pallas import tpu_sc as plsc

N = 10000
E = 320000
D = 128
DE = 16
G = 16
NC = 2
NS = 16
CHUNK = 80
E_PER_TILE = E // (NC * NS)
CHUNKS_PER_TILE = E_PER_TILE // CHUNK
AGG_N = 10240
ROWS_PER_TILE = AGG_N // NS
STAGE = 128
NBLK = 10
BLK = N // NBLK


def _precompute_body(x_ref, batch_ref, u_ref, w1_ref, b1_ref, a_ref, b_ref):
    xb = x_ref[...]
    wu1 = jnp.dot(u_ref[...], w1_ref[2 * D + DE:, :],
                  preferred_element_type=jnp.float32)
    bb = batch_ref[0, 0, :]
    onehot = (bb[:, None] == lax.broadcasted_iota(jnp.int32, (BLK, G), 1)
              ).astype(jnp.float32)
    a_ref[...] = (jnp.dot(xb, w1_ref[0:D, :], preferred_element_type=jnp.float32)
                  + jnp.dot(onehot, wu1, preferred_element_type=jnp.float32)
                  + b1_ref[...])
    b_ref[...] = jnp.dot(xb, w1_ref[D:2 * D, :],
                         preferred_element_type=jnp.float32)


_precompute = pl.pallas_call(
    _precompute_body,
    grid=(NBLK,),
    in_specs=[
        pl.BlockSpec((BLK, D), lambda i: (i, 0)),
        pl.BlockSpec((1, 1, BLK), lambda i: (i, 0, 0)),
        pl.BlockSpec((G, D), lambda i: (0, 0)),
        pl.BlockSpec((2 * D + DE + D, D), lambda i: (0, 0)),
        pl.BlockSpec((1, D), lambda i: (0, 0)),
    ],
    out_specs=[pl.BlockSpec((BLK, D), lambda i: (i, 0))] * 2,
    out_shape=[jax.ShapeDtypeStruct((N, D), jnp.float32)] * 2,
)


def _edge_c_body(ea_ref, we_ref, c_ref):
    c_ref[...] = jnp.dot(ea_ref[...], we_ref[...],
                         preferred_element_type=jnp.float32)


_EBLK = 8000
_edge_c = pl.pallas_call(
    _edge_c_body,
    grid=(E // _EBLK,),
    in_specs=[
        pl.BlockSpec((_EBLK, DE), lambda i: (i, 0)),
        pl.BlockSpec((DE, D), lambda i: (0, 0)),
    ],
    out_specs=pl.BlockSpec((_EBLK, D), lambda i: (i, 0)),
    out_shape=jax.ShapeDtypeStruct((E, D), jnp.float32),
)


@functools.partial(
    pl.kernel,
    mesh=plsc.VectorSubcoreMesh(core_axis_name="c", subcore_axis_name="s"),
    out_type=jax.ShapeDtypeStruct((2 * AGG_N, D), jnp.float32),
    scratch_types=[
        pltpu.VMEM((CHUNK,), jnp.int32),
        pltpu.VMEM((CHUNK,), jnp.int32),
        pltpu.VMEM((CHUNK, D), jnp.float32),
        pltpu.VMEM((CHUNK, D), jnp.float32),
        pltpu.VMEM((CHUNK, D), jnp.float32),
        pltpu.VMEM((STAGE, D), jnp.float32),
        pltpu.VMEM_SHARED((AGG_N, D), jnp.float32),
        pltpu.SemaphoreType.DMA,
        pltpu.SemaphoreType.DMA,
    ],
)
def _sc_agg(a_hbm, b_hbm, c_hbm, src_hbm, dst_hbm, out_hbm,
            si_v, di_v, a_v, b_v, c_v, stage_v, agg_sh, sem_a, sem_b):
    cid = lax.axis_index("c")
    sid = lax.axis_index("s")
    row0 = sid * ROWS_PER_TILE
    zero16 = jnp.zeros((16,), jnp.float32)

    def zero_body(r, carry):
        for s in range(D // 16):
            stage_v[r, pl.ds(s * 16, 16)] = zero16
        return carry

    lax.fori_loop(0, STAGE, zero_body, 0)
    for j in range(ROWS_PER_TILE // STAGE):
        pltpu.sync_copy(stage_v, agg_sh.at[pl.ds(row0 + j * STAGE, STAGE)])
    plsc.subcore_barrier()

    ebase = (cid * NS + sid) * E_PER_TILE

    def chunk_body(t, carry):
        base = pl.multiple_of(ebase + t * CHUNK, 8)
        pltpu.sync_copy(src_hbm.at[pl.ds(base, CHUNK)], si_v)
        pltpu.sync_copy(dst_hbm.at[pl.ds(base, CHUNK)], di_v)
        cp_a = pltpu.async_copy(a_hbm.at[si_v], a_v, sem_a)
        cp_b = pltpu.async_copy(b_hbm.at[di_v], b_v, sem_b)
        pltpu.sync_copy(c_hbm.at[pl.ds(base, CHUNK)], c_v)
        cp_a.wait()
        cp_b.wait()

        def edge_body(e, inner):
            for s in range(D // 16):
                sl = pl.ds(s * 16, 16)
                c_v[e, sl] = jnp.maximum(a_v[e, sl] + b_v[e, sl] + c_v[e, sl],
                                         0.0)
            return inner

        lax.fori_loop(0, CHUNK, edge_body, 0)
        pltpu.sync_copy(c_v, agg_sh.at[di_v], add=True)
        return carry

    lax.fori_loop(0, CHUNKS_PER_TILE, chunk_body, 0)
    plsc.subcore_barrier()

    out0 = cid * AGG_N + row0
    for j in range(ROWS_PER_TILE // STAGE):
        pltpu.sync_copy(agg_sh.at[pl.ds(row0 + j * STAGE, STAGE)], stage_v)
        pltpu.sync_copy(stage_v, out_hbm.at[pl.ds(out0 + j * STAGE, STAGE)])


def _node_out_body(x_ref, p0_ref, p1_ref, batch_ref, u_ref, w2_ref, b2_ref,
                   o_ref):
    xb = x_ref[...]
    agg = p0_ref[...] + p1_ref[...]
    wu2 = jnp.dot(u_ref[...], w2_ref[2 * D:, :],
                  preferred_element_type=jnp.float32)
    bb = batch_ref[0, 0, :]
    onehot = (bb[:, None] == lax.broadcasted_iota(jnp.int32, (BLK, G), 1)
              ).astype(jnp.float32)
    acc = (jnp.dot(xb, w2_ref[0:D, :], preferred_element_type=jnp.float32)
           + jnp.dot(agg, w2_ref[D:2 * D, :], preferred_element_type=jnp.float32)
           + jnp.dot(onehot, wu2, preferred_element_type=jnp.float32)
           + b2_ref[...])
    o_ref[...] = jnp.maximum(acc, 0.0)


_node_out = pl.pallas_call(
    _node_out_body,
    grid=(NBLK,),
    in_specs=[
        pl.BlockSpec((BLK, D), lambda i: (i, 0)),
        pl.BlockSpec((BLK, D), lambda i: (i, 0)),
        pl.BlockSpec((BLK, D), lambda i: (i, 0)),
        pl.BlockSpec((1, 1, BLK), lambda i: (i, 0, 0)),
        pl.BlockSpec((G, D), lambda i: (0, 0)),
        pl.BlockSpec((2 * D + D, D), lambda i: (0, 0)),
        pl.BlockSpec((1, D), lambda i: (0, 0)),
    ],
    out_specs=pl.BlockSpec((BLK, D), lambda i: (i, 0)),
    out_shape=jax.ShapeDtypeStruct((N, D), jnp.float32),
)


def kernel(x, edge_index, edge_attr, u, batch, W1, b1, W2, b2):
    src = edge_index[0].astype(jnp.int32)
    dst = edge_index[1].astype(jnp.int32)
    batch32 = batch.astype(jnp.int32).reshape(NBLK, 1, BLK)
    a_tab, b_tab = _precompute(x, batch32, u, W1, b1.reshape(1, D))
    c_tab = _edge_c(edge_attr, W1[2 * D:2 * D + DE])
    parts = _sc_agg(a_tab, b_tab, c_tab, src, dst)
    p0 = lax.slice(parts, (0, 0), (N, D))
    p1 = lax.slice(parts, (AGG_N, 0), (AGG_N + N, D))
    return _node_out(x, p0, p1, batch32, u, W2, b2.reshape(1, D))
